# baseline (speedup 1.0000x reference)
import functools

import jax
import jax.numpy as jnp
from jax import lax
from jax.experimental import pallas as pl
from jax.experimental.pallas import tpu as pltpu

_K = 10
_ALPHA = 0.1
_TILE = 2000


def _body(x_ref, u_ref, v_ref, w_ref, out_ref, s_ref):
    phase = pl.program_id(0)
    i = pl.program_id(1)

    @pl.when(phase == 0)
    def _accumulate():
        @pl.when(i == 0)
        def _init():
            s_ref[...] = jnp.zeros_like(s_ref)

        s_ref[...] += lax.dot_general(
            u_ref[...], x_ref[...],
            (((0,), (0,)), ((), ())),
            preferred_element_type=jnp.float32,
        )

    @pl.when(phase == 1)
    def _emit():
        v = v_ref[...]
        v_pow = jnp.ones_like(v)
        v_out = jnp.zeros_like(v)
        for _ in range(_K):
            v_pow = v_pow * v
            v_out = v_out + (1.0 - _ALPHA) * v_pow
        v_out = v_out / _K

        uw = u_ref[...] * v_out
        t = lax.dot_general(
            uw, s_ref[...],
            (((1,), (0,)), ((), ())),
            preferred_element_type=jnp.float32,
        ) + _ALPHA * x_ref[...]
        out_ref[...] = lax.dot_general(
            t, w_ref[...],
            (((1,), (1,)), ((), ())),
            preferred_element_type=jnp.float32,
        )


@functools.partial(jax.jit, static_argnames=())
def kernel(x, U, V, W):
    n, d = x.shape
    keig = U.shape[1]
    num_tiles = n // _TILE
    assert num_tiles * _TILE == n

    v2 = V.reshape(1, keig)

    grid = (2, num_tiles)
    out = pl.pallas_call(
        _body,
        grid=grid,
        in_specs=[
            pl.BlockSpec((_TILE, d), lambda p, i: (i, 0)),
            pl.BlockSpec((_TILE, keig), lambda p, i: (i, 0)),
            pl.BlockSpec((1, keig), lambda p, i: (0, 0)),
            pl.BlockSpec((d, d), lambda p, i: (0, 0)),
        ],
        out_specs=pl.BlockSpec((_TILE, d), lambda p, i: (p * i, 0)),
        out_shape=jax.ShapeDtypeStruct((n, d), jnp.float32),
        scratch_shapes=[pltpu.VMEM((keig, d), jnp.float32)],
        compiler_params=pltpu.CompilerParams(
            dimension_semantics=("arbitrary", "arbitrary"),
        ),
    )(x, U, v2, W)
    return out
